# Initial kernel scaffold; baseline (speedup 1.0000x reference)
import jax
import jax.numpy as jnp
from jax.experimental import pallas as pl


def kernel(input_ids, token_table, pos_table):
    raise NotImplementedError("write your pallas kernel here")


# speedup vs baseline: 1.3562x; 1.3562x over previous
import jax
import jax.numpy as jnp
from jax import lax
from jax.experimental import pallas as pl
from jax.experimental.pallas import tpu as pltpu
from jax.experimental.pallas import tpu_sc as plsc

B = 16384
L = 50
DIM = 32
CHUNK = 100
ROWS = (B * L) // CHUNK
NW = 32
CPW = ROWS // NW
NBUF = 8


def _body(idx_hbm, tok_hbm, pos_hbm, out_hbm,
          idx_v, pos_v, r0, r1, r2, r3, r4, r5, r6, r7,
          g0, g1, g2, g3, g4, g5, g6, g7,
          s0, s1, s2, s3, s4, s5, s6, s7):
    rows = (r0, r1, r2, r3, r4, r5, r6, r7)
    gsem = (g0, g1, g2, g3, g4, g5, g6, g7)
    ssem = (s0, s1, s2, s3, s4, s5, s6, s7)

    wid = lax.axis_index("s") * 2 + lax.axis_index("c")
    base = wid * CPW

    pltpu.sync_copy(idx_hbm.at[pl.ds(base, CPW)], idx_v)
    pltpu.sync_copy(pos_hbm, pos_v.at[pl.ds(0, L)])
    pltpu.sync_copy(pos_hbm, pos_v.at[pl.ds(L, L)])

    def start_gather(j, b):
        pltpu.make_async_copy(tok_hbm.at[idx_v.at[j]], rows[b], gsem[b]).start()

    def wait_gather(j, b):
        pltpu.make_async_copy(tok_hbm.at[idx_v.at[j]], rows[b], gsem[b]).wait()

    def start_store(j, b):
        pltpu.make_async_copy(rows[b], out_hbm.at[base + j], ssem[b]).start()

    def wait_store(j, b):
        pltpu.make_async_copy(rows[b], out_hbm.at[base + j], ssem[b]).wait()

    for bi in range(NBUF // 2):
        start_gather(bi, bi)

    def add_pos(b):
        def row_add(r, _):
            plsc.addupdate(rows[b].at[r, pl.ds(0, 16)], pos_v[r, pl.ds(0, 16)])
            plsc.addupdate(rows[b].at[r, pl.ds(16, 16)], pos_v[r, pl.ds(16, 16)])
            return _
        lax.fori_loop(0, CHUNK, row_add, None)

    def step(t, _):
        for bi in range(NBUF):
            j = t * NBUF + bi
            wait_gather(j, bi)
            add_pos(bi)
            start_store(j, bi)
            jn = j + NBUF // 2
            bn = (bi + NBUF // 2) % NBUF

            @pl.when(jn < CPW)
            def _refill():
                @pl.when(j >= NBUF // 2)
                def _drain():
                    wait_store(jn - NBUF, bn)
                start_gather(jn, bn)
        return _

    lax.fori_loop(0, CPW // NBUF, step, None)

    for bi in range(NBUF):
        wait_store(CPW - NBUF + bi, bi)


def kernel(input_ids, token_table, pos_table):
    idx2d = input_ids.astype(jnp.int32).reshape(ROWS, CHUNK)
    mesh = plsc.VectorSubcoreMesh(core_axis_name="c", subcore_axis_name="s")
    out = pl.kernel(
        _body,
        out_type=jax.ShapeDtypeStruct((ROWS, CHUNK, DIM), jnp.float32),
        mesh=mesh,
        compiler_params=pltpu.CompilerParams(use_tc_tiling_on_sc=False),
        scratch_types=(
            [pltpu.VMEM((CPW, CHUNK), jnp.int32),
             pltpu.VMEM((CHUNK, DIM), jnp.float32)]
            + [pltpu.VMEM((CHUNK, DIM), jnp.float32) for _ in range(NBUF)]
            + [pltpu.SemaphoreType.DMA for _ in range(2 * NBUF)]
        ),
    )(idx2d, token_table, pos_table)
    return out.reshape(B, L, DIM)


# speedup vs baseline: 1.5147x; 1.1169x over previous
import jax
import jax.numpy as jnp
from jax import lax
from jax.experimental import pallas as pl
from jax.experimental.pallas import tpu as pltpu
from jax.experimental.pallas import tpu_sc as plsc

B = 16384
L = 50
DIM = 32
CHUNK = 128
NIDX = B * L
NCH = NIDX // CHUNK
NW = 32
CPW = NCH // NW
NBUF = 8
QD = NBUF // 2


def _body(idx_hbm, tok_hbm, pos_hbm, out_hbm,
          idx_v, pos_t, pos_v,
          a0, a1, a2, a3, a4, a5, a6, a7,
          b0, b1, b2, b3, b4, b5, b6, b7,
          g0, g1, g2, g3, g4, g5, g6, g7,
          s0, s1, s2, s3, s4, s5, s6, s7):
    abuf = (a0, a1, a2, a3, a4, a5, a6, a7)
    bbuf = (b0, b1, b2, b3, b4, b5, b6, b7)
    gsem = (g0, g1, g2, g3, g4, g5, g6, g7)
    ssem = (s0, s1, s2, s3, s4, s5, s6, s7)

    wid = lax.axis_index("s") * 2 + lax.axis_index("c")
    base = wid * CPW

    pltpu.sync_copy(idx_hbm.at[pl.ds(base, CPW)], idx_v)
    pltpu.sync_copy(pos_hbm, pos_t)

    def pos_relay(r2, carry):
        for k2 in range(8):
            for c2 in range(2):
                f = 32 * k2 + 16 * c2
                pos_v[8 * r2 + k2, pl.ds(16 * c2, 16)] = (
                    pos_t[2 * r2 + f // 128, pl.ds(f % 128, 16)])
        return carry
    lax.fori_loop(0, 25, pos_relay, None)

    def start_gather(j, b):
        pltpu.make_async_copy(tok_hbm.at[idx_v.at[j]], abuf[b], gsem[b]).start()

    def wait_gather(j, b):
        pltpu.make_async_copy(tok_hbm.at[idx_v.at[j]], abuf[b], gsem[b]).wait()

    def start_store(j, b):
        pltpu.make_async_copy(bbuf[b], out_hbm.at[base + j], ssem[b]).start()

    def wait_store(j, b):
        pltpu.make_async_copy(bbuf[b], out_hbm.at[base + j], ssem[b]).wait()

    for bi in range(QD):
        start_gather(bi, bi)

    def add_pos(b, phase):
        a, bb = abuf[b], bbuf[b]

        def row_add(r, carry):
            for k in range(8):
                c = k % 2
                m = 4 * r + k // 2
                bb[r, pl.ds(16 * k, 16)] = (
                    a[m, pl.ds(16 * c, 16)]
                    + pos_v[phase + m, pl.ds(16 * c, 16)])
            return carry
        lax.fori_loop(0, CHUNK // 4, row_add, None)

    def step(t, carry):
        for bi in range(NBUF):
            j = t * NBUF + bi
            wait_gather(j, bi)
            jn = j + QD
            bn = (bi + QD) % NBUF

            @pl.when(jn < CPW)
            def _refill():
                start_gather(jn, bn)

            @pl.when(j >= NBUF)
            def _drain():
                wait_store(j - NBUF, bi)
            add_pos(bi, lax.rem(28 * j, 50))
            start_store(j, bi)
        return carry

    lax.fori_loop(0, CPW // NBUF, step, None)

    for bi in range(NBUF):
        wait_store(CPW - NBUF + bi, bi)


def kernel(input_ids, token_table, pos_table):
    idx128 = input_ids.astype(jnp.int32).reshape(NCH, CHUNK)
    pos4 = jnp.tile(pos_table, (4, 1)).reshape(L, 128)
    mesh = plsc.VectorSubcoreMesh(core_axis_name="c", subcore_axis_name="s")
    out = pl.kernel(
        _body,
        out_type=jax.ShapeDtypeStruct((NCH, DIM, 128), jnp.float32),
        mesh=mesh,
        compiler_params=pltpu.CompilerParams(use_tc_tiling_on_sc=False),
        scratch_types=(
            [pltpu.VMEM((CPW, CHUNK), jnp.int32),
             pltpu.VMEM((L, 128), jnp.float32),
             pltpu.VMEM((4 * L, DIM), jnp.float32)]
            + [pltpu.VMEM((CHUNK, DIM), jnp.float32) for _ in range(NBUF)]
            + [pltpu.VMEM((DIM, 128), jnp.float32) for _ in range(NBUF)]
            + [pltpu.SemaphoreType.DMA for _ in range(2 * NBUF)]
        ),
    )(idx128, token_table, pos4)
    return out.reshape(B, L, DIM)


# speedup vs baseline: 2.5029x; 1.6525x over previous
import jax
import jax.numpy as jnp
from jax import lax
from jax.experimental import pallas as pl
from jax.experimental.pallas import tpu as pltpu
from jax.experimental.pallas import tpu_sc as plsc

B = 16384
L = 50
DIM = 32
VOCAB = 1000000
CHUNK = 128
NB = B // CHUNK
NCH = L * NB
NW = 32
CPW = NCH // NW
NBUF = 8
QD = NBUF // 2


def _body(idx_hbm, tok_hbm, pos_hbm, out_hbm,
          idx_v, pos_v,
          a0, a1, a2, a3, a4, a5, a6, a7,
          g0, g1, g2, g3, g4, g5, g6, g7,
          s0, s1, s2, s3, s4, s5, s6, s7):
    abuf = (a0, a1, a2, a3, a4, a5, a6, a7)
    gsem = (g0, g1, g2, g3, g4, g5, g6, g7)
    ssem = (s0, s1, s2, s3, s4, s5, s6, s7)

    wid = lax.axis_index("s") * 2 + lax.axis_index("c")
    base = wid * CPW
    l0 = lax.shift_right_logical(base, 7)

    pltpu.sync_copy(idx_hbm.at[pl.ds(base, CPW)], idx_v)
    pltpu.sync_copy(pos_hbm.at[pl.ds(l0, 4)], pos_v)

    def start_gather(j, b):
        pltpu.make_async_copy(tok_hbm.at[idx_v.at[j]], abuf[b], gsem[b]).start()

    def wait_gather(j, b):
        pltpu.make_async_copy(tok_hbm.at[idx_v.at[j]], abuf[b], gsem[b]).wait()

    def chunk_lbb(j):
        c = base + j
        return lax.shift_right_logical(c, 7), lax.bitwise_and(c, 127)

    def start_store(j, b):
        l, bb = chunk_lbb(j)
        for g in range(16):
            pltpu.make_async_copy(
                abuf[b].at[pl.ds(8 * g, 8)],
                out_hbm.at[l, bb * 16 + g, :, pl.ds(0, DIM)],
                ssem[b]).start()

    def wait_store(j, b):
        l, bb = chunk_lbb(j)
        for g in range(16):
            pltpu.make_async_copy(
                abuf[b].at[pl.ds(8 * g, 8)],
                out_hbm.at[l, bb * 16 + g, :, pl.ds(0, DIM)],
                ssem[b]).wait()

    for bi in range(QD):
        start_gather(bi, bi)

    def add_pos(j, b):
        l, _ = chunk_lbb(j)
        lblk = l - l0
        a = abuf[b]
        p0 = pos_v[lblk, pl.ds(0, 16)]
        p1 = pos_v[lblk, pl.ds(16, 16)]

        def rowfn(t, carry):
            a[t, pl.ds(0, 16)] = a[t, pl.ds(0, 16)] + p0
            a[t, pl.ds(16, 16)] = a[t, pl.ds(16, 16)] + p1
            return carry
        lax.fori_loop(0, CHUNK, rowfn, None)

    def step(t, carry):
        for bi in range(NBUF):
            j = t * NBUF + bi
            wait_gather(j, bi)
            add_pos(j, bi)
            start_store(j, bi)
            jn = j + QD
            bn = (bi + QD) % NBUF

            @pl.when(jnp.logical_and(jn < CPW, j >= QD))
            def _drain():
                wait_store(j - QD, bn)

            @pl.when(jn < CPW)
            def _refill():
                start_gather(jn, bn)
        return carry

    lax.fori_loop(0, CPW // NBUF, step, None)

    for bi in range(NBUF):
        wait_store(CPW - NBUF + bi, bi)


def kernel(input_ids, token_table, pos_table):
    idx4 = input_ids.astype(jnp.int32).T.reshape(NCH, CHUNK) * 4
    tokp = jnp.pad(token_table, ((0, 0), (0, 96))).reshape(4 * VOCAB, DIM)
    posp = jnp.pad(pos_table, ((0, 6), (0, 0)))
    mesh = plsc.VectorSubcoreMesh(core_axis_name="c", subcore_axis_name="s")
    out = pl.kernel(
        _body,
        out_type=jax.ShapeDtypeStruct((L, B // 8, 8, 128), jnp.float32),
        mesh=mesh,
        compiler_params=pltpu.CompilerParams(use_tc_tiling_on_sc=False),
        scratch_types=(
            [pltpu.VMEM((CPW, CHUNK), jnp.int32),
             pltpu.VMEM((4, DIM), jnp.float32)]
            + [pltpu.VMEM((CHUNK, DIM), jnp.float32) for _ in range(NBUF)]
            + [pltpu.SemaphoreType.DMA for _ in range(2 * NBUF)]
        ),
    )(idx4, tokp, posp)
    return out.reshape(L, B, 128)[:, :, :DIM].transpose(1, 0, 2)
